# baseline (speedup 1.0000x reference)
import numpy as np
import jax
import jax.numpy as jnp
from jax import lax
from jax.experimental import pallas as pl
from jax.experimental.pallas import tpu as pltpu
from jax.experimental.pallas import tpu_sc as plsc

_B = 32
_N = 8192
_L = 16
_NCH = _N // _L
_RADIX = 256
_HIST = _RADIX * _L


def _tf2x32_np(k1, k2, x0, x1):
    k1 = np.uint32(k1); k2 = np.uint32(k2)
    x0 = x0.astype(np.uint32).copy(); x1 = x1.astype(np.uint32).copy()
    rot = ([np.uint32(r) for r in (13, 15, 26, 6)],
           [np.uint32(r) for r in (17, 29, 16, 24)])
    ks = [k1, k2, np.uint32(k1 ^ k2 ^ np.uint32(0x1BD11BDA))]

    def rotl(v, d):
        return (v << d) | (v >> np.uint32(32 - d))

    x0 = x0 + ks[0]; x1 = x1 + ks[1]
    seq = [(rot[0], ks[1], ks[2], 1), (rot[1], ks[2], ks[0], 2),
           (rot[0], ks[0], ks[1], 3), (rot[1], ks[1], ks[2], 4),
           (rot[0], ks[2], ks[0], 5)]
    for rs, ka, kb, c in seq:
        for r in rs:
            x0 = x0 + x1
            x1 = rotl(x1, r)
            x1 = x1 ^ x0
        x0 = x0 + ka
        x1 = x1 + kb + np.uint32(c)
    return x0, x1


def _split_np(kd, num):
    c1 = np.zeros(num, np.uint32)
    c2 = np.arange(num, dtype=np.uint32)
    b1, b2 = _tf2x32_np(kd[0], kd[1], c1, c2)
    return np.stack([b1, b2], axis=1)


def _subkey_table():
    root = np.array([0, 42], np.uint32)
    bkeys = _split_np(root, _B)
    rows = []
    for b in range(_B):
        kb1, sub1 = _split_np(bkeys[b], 2)
        _, sub2 = _split_np(kb1, 2)
        vals = np.concatenate([np.repeat(sub1, _L), np.repeat(sub2, _L)])
        rows.append(vals)
    return np.stack(rows).astype(np.int64).astype(np.uint32).view(np.int32)


_SUBK = _subkey_table()


def _tf_bits(k1, k2, cnt):
    c = jnp.int32(0x1BD11BDA)
    ks = (k1, k2, k1 ^ k2 ^ c)

    def rotl(v, r):
        return lax.shift_left(v, jnp.int32(r)) | lax.shift_right_logical(
            v, jnp.int32(32 - r))

    x0 = ks[0]
    x1 = cnt + ks[1]
    rots = ((13, 15, 26, 6), (17, 29, 16, 24))
    for i in range(5):
        for r in rots[i % 2]:
            x0 = x0 + x1
            x1 = rotl(x1, r)
            x1 = x1 ^ x0
        x0 = x0 + ks[(i + 1) % 3]
        x1 = x1 + ks[(i + 2) % 3] + jnp.int32(i + 1)
    return x0 ^ x1


def _sc_body(subk_hbm, pa_hbm, re_hbm, sv, ka, kb, pa, pb, hist):
    wid = lax.axis_index("s") * 2 + lax.axis_index("c")
    pltpu.sync_copy(subk_hbm.at[wid], sv)
    lane = lax.iota(jnp.int32, _L)
    lane_nch = lane * _NCH

    def phys(pos):
        return lax.shift_left(pos & jnp.int32(_NCH - 1), jnp.int32(4)) | (
            lax.shift_right_logical(pos, jnp.int32(9)))

    def gen_keys(koff):
        k1 = sv[pl.ds(koff, _L)]
        k2 = sv[pl.ds(koff + _L, _L)]

        def gen(tt, _):
            for u in range(4):
                t = tt * 4 + u
                ka[pl.ds(t * _L, _L)] = _tf_bits(k1, k2, lane_nch + t)
            return 0

        lax.fori_loop(0, _NCH // 4, gen, 0)

    def radix_pass(src_k, src_p, dst_k, dst_p, shift, first, store_keys,
                   final):
        zeros = jnp.zeros((_L,), jnp.int32)
        ones = jnp.ones((_L,), jnp.int32)
        one = jnp.int32(1)
        zero = jnp.int32(0)
        sh = jnp.int32(shift)
        mask = jnp.int32(0xFF)

        def z(jj, _):
            for u in range(8):
                hist[pl.ds((jj * 8 + u) * _L, _L)] = zeros
            return 0

        lax.fori_loop(0, _HIST // _L // 8, z, 0)

        def cnt(tt, _):
            ks = [src_k[pl.ds((tt * 8 + u) * _L, _L)] for u in range(8)]
            hs = [(lax.shift_right_logical(k, sh) & mask) * _L + lane
                  for k in ks]
            for h in hs:
                plsc.addupdate_scatter(hist, [h], ones)
            return 0

        lax.fori_loop(0, _NCH // 8, cnt, 0)

        def scn(jj, carry):
            vs = [hist[pl.ds((jj * 4 + u) * _L, _L)] for u in range(4)]
            incs = [plsc.cumsum(v) for v in vs]
            for u in range(4):
                hist[pl.ds((jj * 4 + u) * _L, _L)] = incs[u] - vs[u] + carry
                carry = carry + incs[u][15]
            return carry

        lax.fori_loop(0, _HIST // _L // 4, scn, jnp.int32(0))

        U = 4

        def prm(tt, _):
            ks, ps, hs = [], [], []
            for u in range(U):
                t = tt * U + u
                ks.append(src_k[pl.ds(t * _L, _L)])
                ps.append((lane_nch + t) if first
                          else src_p[pl.ds(t * _L, _L)])
                hs.append((lax.shift_right_logical(ks[u], sh) & mask) * _L
                          + lane)
            bases = [plsc.load_gather(hist, [h]) for h in hs]
            for h in hs:
                plsc.addupdate_scatter(hist, [h], ones)
            poss = []
            for u in range(U):
                pos = bases[u]
                for up in range(u):
                    pos = pos + jnp.where(hs[up] == hs[u], one, zero)
                poss.append(pos)
            for u in range(U):
                wpos = poss[u] if final else phys(poss[u])
                if store_keys:
                    plsc.store_scatter(dst_k, [wpos], ks[u])
                plsc.store_scatter(dst_p, [wpos], ps[u])
            return 0

        lax.fori_loop(0, _NCH // U, prm, 0)

    with jax.named_scope("gen1"):
        gen_keys(0)
    with jax.named_scope("sort1"):
        radix_pass(ka, None, kb, pb, 0, True, True, False)
        radix_pass(kb, pb, ka, pa, 8, False, True, False)
        radix_pass(ka, pa, kb, pb, 16, False, True, False)
        radix_pass(kb, pb, ka, pa, 24, False, False, False)
    with jax.named_scope("gen2"):
        gen_keys(_L * 2)
    with jax.named_scope("sort2"):
        radix_pass(ka, pa, kb, pb, 0, False, True, False)
        radix_pass(kb, pb, ka, pa, 8, False, True, False)
        radix_pass(ka, pa, kb, pb, 16, False, True, False)
        radix_pass(kb, pb, ka, pa, 24, False, False, True)

    def inv(tt, _):
        vs = [pa[pl.ds((tt * 8 + u) * _L, _L)] for u in range(8)]
        for u in range(8):
            plsc.store_scatter(kb, [vs[u]], lane + (tt * 8 + u) * _L)
        return 0

    lax.fori_loop(0, _NCH // 8, inv, 0)
    pltpu.sync_copy(pa, pa_hbm.at[wid])
    pltpu.sync_copy(kb, re_hbm.at[wid])


def _make_kernel(interpret=False):
    mesh = plsc.VectorSubcoreMesh(core_axis_name="c", subcore_axis_name="s",
                                  num_cores=2, num_subcores=16)
    return pl.kernel(
        _sc_body,
        out_type=(jax.ShapeDtypeStruct((_B, _N), jnp.int32),
                  jax.ShapeDtypeStruct((_B, _N), jnp.int32)),
        mesh=mesh,
        compiler_params=pltpu.CompilerParams(needs_layout_passes=False),
        scratch_types=[
            pltpu.VMEM((4 * _L,), jnp.int32),
            pltpu.VMEM((_N,), jnp.int32),
            pltpu.VMEM((_N,), jnp.int32),
            pltpu.VMEM((_N,), jnp.int32),
            pltpu.VMEM((_N,), jnp.int32),
            pltpu.VMEM((_HIST,), jnp.int32),
        ],
        interpret=interpret,
    )


def kernel(z):
    del z
    idx_pa, idx_re = _make_kernel()(jnp.asarray(_SUBK))
    return idx_pa, idx_re


# speedup vs baseline: 1.0398x; 1.0398x over previous
import numpy as np
import jax
import jax.numpy as jnp
from jax import lax
from jax.experimental import pallas as pl
from jax.experimental.pallas import tpu as pltpu
from jax.experimental.pallas import tpu_sc as plsc

_B = 32
_N = 8192
_L = 16
_NCH = _N // _L
_RADIX = 256
_HIST = _RADIX * _L


def _tf2x32_np(k1, k2, x0, x1):
    k1 = np.uint32(k1); k2 = np.uint32(k2)
    x0 = x0.astype(np.uint32).copy(); x1 = x1.astype(np.uint32).copy()
    rot = ([np.uint32(r) for r in (13, 15, 26, 6)],
           [np.uint32(r) for r in (17, 29, 16, 24)])
    ks = [k1, k2, np.uint32(k1 ^ k2 ^ np.uint32(0x1BD11BDA))]

    def rotl(v, d):
        return (v << d) | (v >> np.uint32(32 - d))

    x0 = x0 + ks[0]; x1 = x1 + ks[1]
    seq = [(rot[0], ks[1], ks[2], 1), (rot[1], ks[2], ks[0], 2),
           (rot[0], ks[0], ks[1], 3), (rot[1], ks[1], ks[2], 4),
           (rot[0], ks[2], ks[0], 5)]
    for rs, ka, kb, c in seq:
        for r in rs:
            x0 = x0 + x1
            x1 = rotl(x1, r)
            x1 = x1 ^ x0
        x0 = x0 + ka
        x1 = x1 + kb + np.uint32(c)
    return x0, x1


def _split_np(kd, num):
    c1 = np.zeros(num, np.uint32)
    c2 = np.arange(num, dtype=np.uint32)
    b1, b2 = _tf2x32_np(kd[0], kd[1], c1, c2)
    return np.stack([b1, b2], axis=1)


def _subkey_table():
    root = np.array([0, 42], np.uint32)
    bkeys = _split_np(root, _B)
    rows = []
    for b in range(_B):
        kb1, sub1 = _split_np(bkeys[b], 2)
        _, sub2 = _split_np(kb1, 2)
        vals = np.concatenate([np.repeat(sub1, _L), np.repeat(sub2, _L)])
        rows.append(vals)
    return np.stack(rows).astype(np.int64).astype(np.uint32).view(np.int32)


_SUBK = _subkey_table()


def _tf_bits(k1, k2, cnt):
    c = jnp.int32(0x1BD11BDA)
    ks = (k1, k2, k1 ^ k2 ^ c)

    def rotl(v, r):
        return lax.shift_left(v, jnp.int32(r)) | lax.shift_right_logical(
            v, jnp.int32(32 - r))

    x0 = ks[0]
    x1 = cnt + ks[1]
    rots = ((13, 15, 26, 6), (17, 29, 16, 24))
    for i in range(5):
        for r in rots[i % 2]:
            x0 = x0 + x1
            x1 = rotl(x1, r)
            x1 = x1 ^ x0
        x0 = x0 + ks[(i + 1) % 3]
        x1 = x1 + ks[(i + 2) % 3] + jnp.int32(i + 1)
    return x0 ^ x1


def _sc_body(subk_hbm, pa_hbm, re_hbm, sv, ka, kb, pa, pb, hist):
    wid = lax.axis_index("s") * 2 + lax.axis_index("c")
    pltpu.sync_copy(subk_hbm.at[wid], sv)
    lane = lax.iota(jnp.int32, _L)
    lane_nch = lane * _NCH

    def phys(pos):
        return lax.shift_left(pos & jnp.int32(_NCH - 1), jnp.int32(4)) | (
            lax.shift_right_logical(pos, jnp.int32(9)))

    def gen_keys(koff):
        k1 = sv[pl.ds(koff, _L)]
        k2 = sv[pl.ds(koff + _L, _L)]

        def gen(tt, _):
            for u in range(4):
                t = tt * 4 + u
                ka[pl.ds(t * _L, _L)] = _tf_bits(k1, k2, lane_nch + t)
            return 0

        lax.fori_loop(0, _NCH // 4, gen, 0)

    def radix_pass(src_k, src_p, dst_k, dst_p, shift, first, store_keys,
                   final):
        zeros = jnp.zeros((_L,), jnp.int32)
        ones = jnp.ones((_L,), jnp.int32)
        one = jnp.int32(1)
        zero = jnp.int32(0)
        sh = jnp.int32(shift)
        mask = jnp.int32(0xFF)

        def z(jj, _):
            for u in range(8):
                hist[pl.ds((jj * 8 + u) * _L, _L)] = zeros
            return 0

        lax.fori_loop(0, _HIST // _L // 8, z, 0)

        def cnt(tt, _):
            ks = [src_k[pl.ds((tt * 8 + u) * _L, _L)] for u in range(8)]
            hs = [(lax.shift_right_logical(k, sh) & mask) * _L + lane
                  for k in ks]
            for h in hs:
                plsc.addupdate_scatter(hist, [h], ones)
            return 0

        lax.fori_loop(0, _NCH // 8, cnt, 0)

        def scn(jj, carry):
            vs = [hist[pl.ds((jj * 8 + u) * _L, _L)] for u in range(8)]
            incs = [plsc.cumsum(v) for v in vs]
            for u in range(8):
                hist[pl.ds((jj * 8 + u) * _L, _L)] = incs[u] - vs[u] + carry
                carry = carry + incs[u][15]
            return carry

        lax.fori_loop(0, _HIST // _L // 8, scn, jnp.int32(0))

        U = 4

        def prm(tt, _):
            ks, ps, hs = [], [], []
            for u in range(U):
                t = tt * U + u
                ks.append(src_k[pl.ds(t * _L, _L)])
                ps.append((lane_nch + t) if first
                          else src_p[pl.ds(t * _L, _L)])
                hs.append((lax.shift_right_logical(ks[u], sh) & mask) * _L
                          + lane)
            bases = [plsc.load_gather(hist, [h]) for h in hs]
            for h in hs:
                plsc.addupdate_scatter(hist, [h], ones)
            poss = []
            for u in range(U):
                pos = bases[u]
                for up in range(u):
                    pos = pos + jnp.where(hs[up] == hs[u], one, zero)
                poss.append(pos)
            for u in range(U):
                wpos = poss[u] if final else phys(poss[u])
                if store_keys:
                    plsc.store_scatter(dst_k, [wpos], ks[u])
                plsc.store_scatter(dst_p, [wpos], ps[u])
            return 0

        lax.fori_loop(0, _NCH // U, prm, 0)

    with jax.named_scope("gen1"):
        gen_keys(0)
    with jax.named_scope("sort1"):
        radix_pass(ka, None, kb, pb, 0, True, True, False)
        radix_pass(kb, pb, ka, pa, 8, False, True, False)
        radix_pass(ka, pa, kb, pb, 16, False, True, False)
        radix_pass(kb, pb, ka, pa, 24, False, False, False)
    with jax.named_scope("gen2"):
        gen_keys(_L * 2)
    with jax.named_scope("sort2"):
        radix_pass(ka, pa, kb, pb, 0, False, True, False)
        radix_pass(kb, pb, ka, pa, 8, False, True, False)
        radix_pass(ka, pa, kb, pb, 16, False, True, False)
        radix_pass(kb, pb, ka, pa, 24, False, False, True)

    def inv(tt, _):
        vs = [pa[pl.ds((tt * 8 + u) * _L, _L)] for u in range(8)]
        for u in range(8):
            plsc.store_scatter(kb, [vs[u]], lane + (tt * 8 + u) * _L)
        return 0

    lax.fori_loop(0, _NCH // 8, inv, 0)
    pltpu.sync_copy(pa, pa_hbm.at[wid])
    pltpu.sync_copy(kb, re_hbm.at[wid])


def _make_kernel(interpret=False):
    mesh = plsc.VectorSubcoreMesh(core_axis_name="c", subcore_axis_name="s",
                                  num_cores=2, num_subcores=16)
    return pl.kernel(
        _sc_body,
        out_type=(jax.ShapeDtypeStruct((_B, _N), jnp.int32),
                  jax.ShapeDtypeStruct((_B, _N), jnp.int32)),
        mesh=mesh,
        compiler_params=pltpu.CompilerParams(needs_layout_passes=False),
        scratch_types=[
            pltpu.VMEM((4 * _L,), jnp.int32),
            pltpu.VMEM((_N,), jnp.int32),
            pltpu.VMEM((_N,), jnp.int32),
            pltpu.VMEM((_N,), jnp.int32),
            pltpu.VMEM((_N,), jnp.int32),
            pltpu.VMEM((_HIST,), jnp.int32),
        ],
        interpret=interpret,
    )


def kernel(z):
    del z
    idx_pa, idx_re = _make_kernel()(jnp.asarray(_SUBK))
    return idx_pa, idx_re
